# Initial kernel scaffold; baseline (speedup 1.0000x reference)
import jax
import jax.numpy as jnp
from jax.experimental import pallas as pl


def kernel(pcd, W, b, gamma, beta):
    raise NotImplementedError("write your pallas kernel here")


# speedup vs baseline: 3.3372x; 3.3372x over previous
import functools

import jax
import jax.numpy as jnp
from jax import lax
from jax.experimental import pallas as pl
from jax.experimental.pallas import tpu as pltpu

_N = 10000
_K = 20
_ROWS = 200
_NPAD = 10112


def _knn_stats_body(rows, k, npad, q_ref, post_ref, sq_ref, stats_ref):
    i = pl.program_id(0)
    q = q_ref[...]
    post = post_ref[...]
    sq = sq_ref[...]

    b = jnp.dot(q, post, preferred_element_type=jnp.float32)
    d = (q[:, 3:4] - 2.0 * b) + sq

    iota = lax.broadcasted_iota(jnp.int32, (rows, npad), 1)

    def extract(_, dcur):
        m = jnp.min(dcur, axis=1, keepdims=True)
        im = jnp.min(jnp.where(dcur == m, iota, npad), axis=1, keepdims=True)
        return jnp.where(iota == im, jnp.inf, dcur)

    d = lax.fori_loop(0, k, extract, d)
    sel = d == jnp.inf

    fq = q[:, 4:5]
    cnt = jnp.sum(jnp.where(sel, 1.0, 0.0), axis=0, keepdims=True)
    sfq = jnp.sum(jnp.where(sel, fq, 0.0), axis=0, keepdims=True)
    mx = jnp.max(jnp.where(sel, fq, -jnp.inf), axis=0, keepdims=True)
    mn = jnp.min(jnp.where(sel, fq, jnp.inf), axis=0, keepdims=True)

    @pl.when(i == 0)
    def _():
        stats_ref[...] = jnp.concatenate([
            jnp.zeros((2, npad), jnp.float32),
            jnp.full((1, npad), -jnp.inf, jnp.float32),
            jnp.full((1, npad), jnp.inf, jnp.float32),
            jnp.zeros((4, npad), jnp.float32),
        ], axis=0)

    stats_ref[0:1, :] = stats_ref[0:1, :] + cnt
    stats_ref[1:2, :] = stats_ref[1:2, :] + sfq
    stats_ref[2:3, :] = jnp.maximum(stats_ref[2:3, :], mx)
    stats_ref[3:4, :] = jnp.minimum(stats_ref[3:4, :], mn)


def _make_knn_stats(n, k, rows, npad, interpret=False):
    body = functools.partial(_knn_stats_body, rows, k, npad)
    return pl.pallas_call(
        body,
        grid=(n // rows,),
        in_specs=[
            pl.BlockSpec((rows, 8), lambda i: (i, 0)),
            pl.BlockSpec((8, npad), lambda i: (0, 0)),
            pl.BlockSpec((1, npad), lambda i: (0, 0)),
        ],
        out_specs=pl.BlockSpec((8, npad), lambda i: (0, 0)),
        out_shape=jax.ShapeDtypeStruct((8, npad), jnp.float32),
        interpret=interpret,
    )


def _expand_body(s_ref, c_ref, o_ref):
    f = s_ref[:, 0:1]
    cnt = s_ref[:, 1:2]
    mx = s_ref[:, 2:3]
    mn = s_ref[:, 3:4]
    pc = c_ref[0:1, :]
    qc = c_ref[1:2, :]
    rc = c_ref[2:3, :]
    t = f * pc + rc + jnp.where(qc >= 0.0, qc * mx, qc * mn)
    y = jnp.where(t >= 0.0, t, 0.2 * t)
    o_ref[...] = jnp.where(cnt > 0.5, y, -jnp.inf)


def _make_expand(n, rows, interpret=False):
    return pl.pallas_call(
        _expand_body,
        grid=(n // rows,),
        in_specs=[
            pl.BlockSpec((rows, 8), lambda i: (i, 0)),
            pl.BlockSpec((8, 64), lambda i: (0, 0)),
        ],
        out_specs=pl.BlockSpec((rows, 64), lambda i: (i, 0)),
        out_shape=jax.ShapeDtypeStruct((n, 64), jnp.float32),
        interpret=interpret,
    )


def _edge_conv(pcd, W, b, gamma, beta, n, k, rows, npad, erows,
               interpret=False):
    f32 = jnp.float32
    pos = pcd[:, :3].astype(f32)
    f = pcd[:, 3].astype(f32)
    qsq = jnp.sum(pos * pos, axis=1)

    qmat = jnp.zeros((n, 8), f32)
    qmat = qmat.at[:, :3].set(pos).at[:, 3].set(qsq).at[:, 4].set(f)
    post = jnp.zeros((8, npad), f32).at[:3, :n].set(pos.T)
    sq = jnp.full((1, npad), 1e30, f32).at[0, :n].set(qsq)

    stats = _make_knn_stats(n, k, rows, npad, interpret)(qmat, post, sq)
    cnt = stats[0, :n]
    sfq = stats[1, :n]
    mx = stats[2, :n]
    mn = stats[3, :n]

    e = float(n * k)
    sum_fp = jnp.dot(cnt, f)
    sum_fp2 = jnp.dot(cnt, f * f)
    sum_fpfq = jnp.dot(sfq, f)
    sum_fq = float(k) * jnp.sum(f)
    sum_fq2 = float(k) * jnp.sum(f * f)
    m_fp = sum_fp / e
    m_fq = sum_fq / e
    v_fp = sum_fp2 / e - m_fp * m_fp
    v_fq = sum_fq2 / e - m_fq * m_fq
    c_pq = sum_fpfq / e - m_fp * m_fq

    w0 = W[0].astype(f32)
    w1 = W[1].astype(f32)
    a0 = w0 - w1
    var = a0 * a0 * v_fp + 2.0 * a0 * w1 * c_pq + w1 * w1 * v_fq
    sigma = jnp.sqrt(var + 1e-5)
    g = gamma.astype(f32) / sigma
    pcoef = a0 * g
    qcoef = w1 * g
    rcoef = beta.astype(f32) - (m_fp * a0 + m_fq * w1) * g

    coefs = jnp.zeros((8, 64), f32)
    coefs = coefs.at[0].set(pcoef).at[1].set(qcoef).at[2].set(rcoef)
    smat = jnp.zeros((n, 8), f32)
    smat = smat.at[:, 0].set(f).at[:, 1].set(cnt).at[:, 2].set(mx)
    smat = smat.at[:, 3].set(mn)

    return _make_expand(n, erows, interpret)(smat, coefs)


def kernel(pcd, W, b, gamma, beta):
    return _edge_conv(pcd, W, b, gamma, beta, _N, _K, _ROWS, _NPAD, 1000)


# speedup vs baseline: 3.5029x; 1.0496x over previous
import functools

import jax
import jax.numpy as jnp
from jax import lax
from jax.experimental import pallas as pl
from jax.experimental.pallas import tpu as pltpu

_N = 10000
_K = 20
_ROWS = 200
_NPAD = 10112


def _knn_stats_body(rows, k, npad, q_ref, post_ref, sq_ref, stats_ref):
    i = pl.program_id(0)
    q = q_ref[...]
    post = post_ref[...]
    sq = sq_ref[...]

    b = jnp.dot(q, post, preferred_element_type=jnp.float32)
    d = (q[:, 3:4] - 2.0 * b) + sq

    iota = lax.broadcasted_iota(jnp.int32, (rows, npad), 1)

    def extract(_, dcur):
        im = jnp.argmin(dcur, axis=1).astype(jnp.int32).reshape(rows, 1)
        return jnp.where(iota == im, jnp.inf, dcur)

    d = lax.fori_loop(0, k, extract, d)
    sel = d == jnp.inf

    fq = q[:, 4:5]
    cnt = jnp.sum(jnp.where(sel, 1.0, 0.0), axis=0, keepdims=True)
    sfq = jnp.sum(jnp.where(sel, fq, 0.0), axis=0, keepdims=True)
    mx = jnp.max(jnp.where(sel, fq, -jnp.inf), axis=0, keepdims=True)
    mn = jnp.min(jnp.where(sel, fq, jnp.inf), axis=0, keepdims=True)

    @pl.when(i == 0)
    def _():
        stats_ref[...] = jnp.concatenate([
            jnp.zeros((2, npad), jnp.float32),
            jnp.full((1, npad), -jnp.inf, jnp.float32),
            jnp.full((1, npad), jnp.inf, jnp.float32),
            jnp.zeros((4, npad), jnp.float32),
        ], axis=0)

    stats_ref[0:1, :] = stats_ref[0:1, :] + cnt
    stats_ref[1:2, :] = stats_ref[1:2, :] + sfq
    stats_ref[2:3, :] = jnp.maximum(stats_ref[2:3, :], mx)
    stats_ref[3:4, :] = jnp.minimum(stats_ref[3:4, :], mn)


def _make_knn_stats(n, k, rows, npad, interpret=False):
    body = functools.partial(_knn_stats_body, rows, k, npad)
    return pl.pallas_call(
        body,
        grid=(n // rows,),
        in_specs=[
            pl.BlockSpec((rows, 8), lambda i: (i, 0)),
            pl.BlockSpec((8, npad), lambda i: (0, 0)),
            pl.BlockSpec((1, npad), lambda i: (0, 0)),
        ],
        out_specs=pl.BlockSpec((8, npad), lambda i: (0, 0)),
        out_shape=jax.ShapeDtypeStruct((8, npad), jnp.float32),
        interpret=interpret,
    )


def _expand_body(s_ref, c_ref, o_ref):
    f = s_ref[:, 0:1]
    cnt = s_ref[:, 1:2]
    mx = s_ref[:, 2:3]
    mn = s_ref[:, 3:4]
    pc = c_ref[0:1, :]
    qc = c_ref[1:2, :]
    rc = c_ref[2:3, :]
    t = f * pc + rc + jnp.where(qc >= 0.0, qc * mx, qc * mn)
    y = jnp.where(t >= 0.0, t, 0.2 * t)
    o_ref[...] = jnp.where(cnt > 0.5, y, -jnp.inf)


def _make_expand(n, rows, interpret=False):
    return pl.pallas_call(
        _expand_body,
        grid=(n // rows,),
        in_specs=[
            pl.BlockSpec((rows, 8), lambda i: (i, 0)),
            pl.BlockSpec((8, 64), lambda i: (0, 0)),
        ],
        out_specs=pl.BlockSpec((rows, 64), lambda i: (i, 0)),
        out_shape=jax.ShapeDtypeStruct((n, 64), jnp.float32),
        interpret=interpret,
    )


def _edge_conv(pcd, W, b, gamma, beta, n, k, rows, npad, erows,
               interpret=False):
    f32 = jnp.float32
    pos = pcd[:, :3].astype(f32)
    f = pcd[:, 3].astype(f32)
    qsq = jnp.sum(pos * pos, axis=1)

    qmat = jnp.zeros((n, 8), f32)
    qmat = qmat.at[:, :3].set(pos).at[:, 3].set(qsq).at[:, 4].set(f)
    post = jnp.zeros((8, npad), f32).at[:3, :n].set(pos.T)
    sq = jnp.full((1, npad), 1e30, f32).at[0, :n].set(qsq)

    stats = _make_knn_stats(n, k, rows, npad, interpret)(qmat, post, sq)
    cnt = stats[0, :n]
    sfq = stats[1, :n]
    mx = stats[2, :n]
    mn = stats[3, :n]

    e = float(n * k)
    sum_fp = jnp.dot(cnt, f)
    sum_fp2 = jnp.dot(cnt, f * f)
    sum_fpfq = jnp.dot(sfq, f)
    sum_fq = float(k) * jnp.sum(f)
    sum_fq2 = float(k) * jnp.sum(f * f)
    m_fp = sum_fp / e
    m_fq = sum_fq / e
    v_fp = sum_fp2 / e - m_fp * m_fp
    v_fq = sum_fq2 / e - m_fq * m_fq
    c_pq = sum_fpfq / e - m_fp * m_fq

    w0 = W[0].astype(f32)
    w1 = W[1].astype(f32)
    a0 = w0 - w1
    var = a0 * a0 * v_fp + 2.0 * a0 * w1 * c_pq + w1 * w1 * v_fq
    sigma = jnp.sqrt(var + 1e-5)
    g = gamma.astype(f32) / sigma
    pcoef = a0 * g
    qcoef = w1 * g
    rcoef = beta.astype(f32) - (m_fp * a0 + m_fq * w1) * g

    coefs = jnp.zeros((8, 64), f32)
    coefs = coefs.at[0].set(pcoef).at[1].set(qcoef).at[2].set(rcoef)
    smat = jnp.zeros((n, 8), f32)
    smat = smat.at[:, 0].set(f).at[:, 1].set(cnt).at[:, 2].set(mx)
    smat = smat.at[:, 3].set(mn)

    return _make_expand(n, erows, interpret)(smat, coefs)


def kernel(pcd, W, b, gamma, beta):
    return _edge_conv(pcd, W, b, gamma, beta, _N, _K, _ROWS, _NPAD, 1000)


# speedup vs baseline: 7.9163x; 2.2599x over previous
import functools

import jax
import jax.numpy as jnp
from jax import lax
from jax.experimental import pallas as pl
from jax.experimental.pallas import tpu as pltpu

_N = 10000
_K = 20
_ROWS = 128
_NQPAD = 10240
_NPAD = 10112


def _knn_stats_body(rows, k, npad, q_ref, post_ref, sq_ref, ofq_ref,
                    stats_ref, d_ref):
    i = pl.program_id(0)
    q = q_ref[...]
    post = post_ref[...]
    sq = sq_ref[...]
    ofq = ofq_ref[...]
    inf = jnp.inf

    bmat = jnp.dot(q, post, preferred_element_type=jnp.float32)
    d_ref[...] = (q[:, 3:4] + bmat) + sq
    fq = q[:, 4:5]
    valid = q[:, 5:6]

    nchunks = npad // 128
    init = tuple(jnp.full((rows, 128), inf, jnp.float32) for _ in range(4))

    def chunk_body(c, carry):
        a1, a2, a3, a4 = carry
        v = d_ref[:, pl.ds(pl.multiple_of(c * 128, 128), 128)]
        a4 = jnp.minimum(a4, v)
        a3, a4 = jnp.minimum(a3, a4), jnp.maximum(a3, a4)
        a2, a3 = jnp.minimum(a2, a3), jnp.maximum(a2, a3)
        a1, a2 = jnp.minimum(a1, a2), jnp.maximum(a1, a2)
        return a1, a2, a3, a4

    a1, a2, a3, a4 = lax.fori_loop(0, nchunks, chunk_body, init)
    cand = jnp.concatenate([a1, a2, a3, a4], axis=1)

    iota_c = lax.broadcasted_iota(jnp.int32, (rows, 512), 1)

    def extract_c(_, cc):
        im = jnp.argmin(cc, axis=1).astype(jnp.int32).reshape(rows, 1)
        return jnp.where(iota_c == im, inf, cc)

    cand = lax.fori_loop(0, k - 1, extract_c, cand)
    t = jnp.min(cand, axis=1, keepdims=True)

    d = d_ref[...]
    u01 = jnp.where((d <= t) & (valid > 0.5), 1.0, 0.0)
    c_le = jnp.sum(u01, axis=1)
    good = jnp.all((c_le == float(k)) | (valid[:, 0] == 0.0))

    @pl.when(i == 0)
    def _():
        stats_ref[...] = jnp.concatenate([
            jnp.zeros((2, npad), jnp.float32),
            jnp.full((1, npad), -jnp.inf, jnp.float32),
            jnp.full((1, npad), jnp.inf, jnp.float32),
            jnp.zeros((4, npad), jnp.float32),
        ], axis=0)

    def accum(sel01):
        cs = jnp.dot(ofq, sel01, preferred_element_type=jnp.float32)
        selm = sel01 > 0.5
        mx = jnp.max(jnp.where(selm, fq, -inf), axis=0, keepdims=True)
        mn = jnp.min(jnp.where(selm, fq, inf), axis=0, keepdims=True)
        stats_ref[0:2, :] = stats_ref[0:2, :] + cs[0:2, :]
        stats_ref[2:3, :] = jnp.maximum(stats_ref[2:3, :], mx)
        stats_ref[3:4, :] = jnp.minimum(stats_ref[3:4, :], mn)

    @pl.when(good)
    def _():
        accum(u01)

    @pl.when(jnp.logical_not(good))
    def _():
        iota = lax.broadcasted_iota(jnp.int32, (rows, npad), 1)

        def extract(_, dcur):
            im = jnp.argmin(dcur, axis=1).astype(jnp.int32).reshape(rows, 1)
            return jnp.where(iota == im, inf, dcur)

        dm = lax.fori_loop(0, k, extract, d)
        accum(jnp.where((dm == inf) & (valid > 0.5), 1.0, 0.0))


def _make_knn_stats(nq, k, rows, npad, interpret=False):
    body = functools.partial(_knn_stats_body, rows, k, npad)
    return pl.pallas_call(
        body,
        grid=(nq // rows,),
        in_specs=[
            pl.BlockSpec((rows, 8), lambda i: (i, 0)),
            pl.BlockSpec((8, npad), lambda i: (0, 0)),
            pl.BlockSpec((1, npad), lambda i: (0, 0)),
            pl.BlockSpec((8, rows), lambda i: (0, i)),
        ],
        out_specs=pl.BlockSpec((8, npad), lambda i: (0, 0)),
        out_shape=jax.ShapeDtypeStruct((8, npad), jnp.float32),
        scratch_shapes=[pltpu.VMEM((rows, npad), jnp.float32)],
        interpret=interpret,
    )


def _expand_body(s_ref, c_ref, o_ref):
    f = s_ref[:, 0:1]
    cnt = s_ref[:, 1:2]
    mx = s_ref[:, 2:3]
    mn = s_ref[:, 3:4]
    pc = c_ref[0:1, :]
    qc = c_ref[1:2, :]
    rc = c_ref[2:3, :]
    t = f * pc + rc + jnp.where(qc >= 0.0, qc * mx, qc * mn)
    y = jnp.where(t >= 0.0, t, 0.2 * t)
    o_ref[...] = jnp.where(cnt > 0.5, y, -jnp.inf)


def _make_expand(n, rows, interpret=False):
    return pl.pallas_call(
        _expand_body,
        grid=(n // rows,),
        in_specs=[
            pl.BlockSpec((rows, 8), lambda i: (i, 0)),
            pl.BlockSpec((8, 64), lambda i: (0, 0)),
        ],
        out_specs=pl.BlockSpec((rows, 64), lambda i: (i, 0)),
        out_shape=jax.ShapeDtypeStruct((n, 64), jnp.float32),
        interpret=interpret,
    )


def _edge_conv(pcd, W, b, gamma, beta, n, k, rows, nqpad, npad, erows,
               interpret=False):
    f32 = jnp.float32
    pos = pcd[:, :3].astype(f32)
    f = pcd[:, 3].astype(f32)
    qsq = jnp.sum(pos * pos, axis=1)

    qmat = jnp.zeros((nqpad, 8), f32)
    qmat = qmat.at[:n, :3].set(pos).at[:n, 3].set(qsq).at[:n, 4].set(f)
    qmat = qmat.at[:n, 5].set(1.0)
    qmat = qmat.at[n:, :5].set(qmat[0, :5])
    post = jnp.zeros((8, npad), f32).at[:3, :n].set(-2.0 * pos.T)
    sq = jnp.full((1, npad), 1e30, f32).at[0, :n].set(qsq)
    ofq = jnp.zeros((8, nqpad), f32).at[0, :].set(1.0).at[1, :n].set(f)

    stats = _make_knn_stats(nqpad, k, rows, npad, interpret)(
        qmat, post, sq, ofq)
    cnt = stats[0, :n]
    sfq = stats[1, :n]
    mx = stats[2, :n]
    mn = stats[3, :n]

    e = float(n * k)
    sum_fp = jnp.dot(cnt, f)
    sum_fp2 = jnp.dot(cnt, f * f)
    sum_fpfq = jnp.dot(sfq, f)
    sum_fq = float(k) * jnp.sum(f)
    sum_fq2 = float(k) * jnp.sum(f * f)
    m_fp = sum_fp / e
    m_fq = sum_fq / e
    v_fp = sum_fp2 / e - m_fp * m_fp
    v_fq = sum_fq2 / e - m_fq * m_fq
    c_pq = sum_fpfq / e - m_fp * m_fq

    w0 = W[0].astype(f32)
    w1 = W[1].astype(f32)
    a0 = w0 - w1
    var = a0 * a0 * v_fp + 2.0 * a0 * w1 * c_pq + w1 * w1 * v_fq
    sigma = jnp.sqrt(var + 1e-5)
    g = gamma.astype(f32) / sigma
    pcoef = a0 * g
    qcoef = w1 * g
    rcoef = beta.astype(f32) - (m_fp * a0 + m_fq * w1) * g

    coefs = jnp.zeros((8, 64), f32)
    coefs = coefs.at[0].set(pcoef).at[1].set(qcoef).at[2].set(rcoef)
    smat = jnp.zeros((n, 8), f32)
    smat = smat.at[:, 0].set(f).at[:, 1].set(cnt).at[:, 2].set(mx)
    smat = smat.at[:, 3].set(mn)

    return _make_expand(n, erows, interpret)(smat, coefs)


def kernel(pcd, W, b, gamma, beta):
    return _edge_conv(pcd, W, b, gamma, beta, _N, _K, _ROWS, _NQPAD, _NPAD,
                      1000)


# speedup vs baseline: 9.9048x; 1.2512x over previous
import functools

import jax
import jax.numpy as jnp
from jax import lax
from jax.experimental import pallas as pl
from jax.experimental.pallas import tpu as pltpu

_N = 10000
_K = 20
_ROWS = 200
_NPAD = 10112


def _knn_stats_body(rows, k, npad, q_ref, post_ref, sq_ref, stats_ref, d_ref):
    i = pl.program_id(0)
    q = q_ref[...]
    post = post_ref[...]
    sq = sq_ref[...]
    inf = jnp.inf

    bmat = jnp.dot(q, post, preferred_element_type=jnp.float32)
    d_ref[...] = (q[:, 3:4] + bmat) + sq
    fq = q[:, 4:5]

    nchunks = npad // 128
    init = tuple(jnp.full((rows, 128), inf, jnp.float32) for _ in range(4))

    def chunk_body(c, carry):
        a1, a2, a3, a4 = carry
        v = d_ref[:, pl.ds(pl.multiple_of(c * 128, 128), 128)]
        a4 = jnp.minimum(a4, v)
        a3, a4 = jnp.minimum(a3, a4), jnp.maximum(a3, a4)
        a2, a3 = jnp.minimum(a2, a3), jnp.maximum(a2, a3)
        a1, a2 = jnp.minimum(a1, a2), jnp.maximum(a1, a2)
        return a1, a2, a3, a4

    a1, a2, a3, a4 = lax.fori_loop(0, nchunks, chunk_body, init)
    cand = jnp.concatenate([a1, a2, a3, a4], axis=1)

    iota_c = lax.broadcasted_iota(jnp.int32, (rows, 512), 1)

    def extract_c(_, cc):
        im = jnp.argmin(cc, axis=1).astype(jnp.int32).reshape(rows, 1)
        return jnp.where(iota_c == im, inf, cc)

    cand = lax.fori_loop(0, k - 1, extract_c, cand)
    t = jnp.min(cand, axis=1, keepdims=True)

    d = d_ref[...]
    le = d <= t
    c_le = jnp.sum(jnp.where(le, 1.0, 0.0), axis=1)
    good = jnp.all(c_le == float(k))

    @pl.when(i == 0)
    def _():
        stats_ref[...] = jnp.concatenate([
            jnp.zeros((2, npad), jnp.float32),
            jnp.full((1, npad), -jnp.inf, jnp.float32),
            jnp.full((1, npad), jnp.inf, jnp.float32),
            jnp.zeros((4, npad), jnp.float32),
        ], axis=0)

    def accum(sel):
        cnt = jnp.sum(jnp.where(sel, 1.0, 0.0), axis=0, keepdims=True)
        sfq = jnp.sum(jnp.where(sel, fq, 0.0), axis=0, keepdims=True)
        mx = jnp.max(jnp.where(sel, fq, -inf), axis=0, keepdims=True)
        mn = jnp.min(jnp.where(sel, fq, inf), axis=0, keepdims=True)
        stats_ref[0:1, :] = stats_ref[0:1, :] + cnt
        stats_ref[1:2, :] = stats_ref[1:2, :] + sfq
        stats_ref[2:3, :] = jnp.maximum(stats_ref[2:3, :], mx)
        stats_ref[3:4, :] = jnp.minimum(stats_ref[3:4, :], mn)

    @pl.when(good)
    def _():
        accum(le)

    @pl.when(jnp.logical_not(good))
    def _():
        iota = lax.broadcasted_iota(jnp.int32, (rows, npad), 1)

        def extract(_, dcur):
            im = jnp.argmin(dcur, axis=1).astype(jnp.int32).reshape(rows, 1)
            return jnp.where(iota == im, inf, dcur)

        dm = lax.fori_loop(0, k, extract, d)
        accum(dm == inf)


def _make_knn_stats(n, k, rows, npad, interpret=False):
    body = functools.partial(_knn_stats_body, rows, k, npad)
    return pl.pallas_call(
        body,
        grid=(n // rows,),
        in_specs=[
            pl.BlockSpec((rows, 8), lambda i: (i, 0)),
            pl.BlockSpec((8, npad), lambda i: (0, 0)),
            pl.BlockSpec((1, npad), lambda i: (0, 0)),
        ],
        out_specs=pl.BlockSpec((8, npad), lambda i: (0, 0)),
        out_shape=jax.ShapeDtypeStruct((8, npad), jnp.float32),
        scratch_shapes=[pltpu.VMEM((rows, npad), jnp.float32)],
        interpret=interpret,
    )


def _expand_body(s_ref, c_ref, o_ref):
    f = s_ref[:, 0:1]
    cnt = s_ref[:, 1:2]
    mx = s_ref[:, 2:3]
    mn = s_ref[:, 3:4]
    pc = c_ref[0:1, :]
    qc = c_ref[1:2, :]
    rc = c_ref[2:3, :]
    t = f * pc + rc + jnp.where(qc >= 0.0, qc * mx, qc * mn)
    y = jnp.where(t >= 0.0, t, 0.2 * t)
    o_ref[...] = jnp.where(cnt > 0.5, y, -jnp.inf)


def _make_expand(n, rows, interpret=False):
    return pl.pallas_call(
        _expand_body,
        grid=(n // rows,),
        in_specs=[
            pl.BlockSpec((rows, 8), lambda i: (i, 0)),
            pl.BlockSpec((8, 64), lambda i: (0, 0)),
        ],
        out_specs=pl.BlockSpec((rows, 64), lambda i: (i, 0)),
        out_shape=jax.ShapeDtypeStruct((n, 64), jnp.float32),
        interpret=interpret,
    )


def _edge_conv(pcd, W, b, gamma, beta, n, k, rows, npad, erows,
               interpret=False):
    f32 = jnp.float32
    pos = pcd[:, :3].astype(f32)
    f = pcd[:, 3].astype(f32)
    qsq = jnp.sum(pos * pos, axis=1)

    qmat = jnp.zeros((n, 8), f32)
    qmat = qmat.at[:, :3].set(pos).at[:, 3].set(qsq).at[:, 4].set(f)
    post = jnp.zeros((8, npad), f32).at[:3, :n].set(-2.0 * pos.T)
    sq = jnp.full((1, npad), 1e30, f32).at[0, :n].set(qsq)

    stats = _make_knn_stats(n, k, rows, npad, interpret)(qmat, post, sq)
    cnt = stats[0, :n]
    sfq = stats[1, :n]
    mx = stats[2, :n]
    mn = stats[3, :n]

    e = float(n * k)
    sum_fp = jnp.dot(cnt, f)
    sum_fp2 = jnp.dot(cnt, f * f)
    sum_fpfq = jnp.dot(sfq, f)
    sum_fq = float(k) * jnp.sum(f)
    sum_fq2 = float(k) * jnp.sum(f * f)
    m_fp = sum_fp / e
    m_fq = sum_fq / e
    v_fp = sum_fp2 / e - m_fp * m_fp
    v_fq = sum_fq2 / e - m_fq * m_fq
    c_pq = sum_fpfq / e - m_fp * m_fq

    w0 = W[0].astype(f32)
    w1 = W[1].astype(f32)
    a0 = w0 - w1
    var = a0 * a0 * v_fp + 2.0 * a0 * w1 * c_pq + w1 * w1 * v_fq
    sigma = jnp.sqrt(var + 1e-5)
    g = gamma.astype(f32) / sigma
    pcoef = a0 * g
    qcoef = w1 * g
    rcoef = beta.astype(f32) - (m_fp * a0 + m_fq * w1) * g

    coefs = jnp.zeros((8, 64), f32)
    coefs = coefs.at[0].set(pcoef).at[1].set(qcoef).at[2].set(rcoef)
    smat = jnp.zeros((n, 8), f32)
    smat = smat.at[:, 0].set(f).at[:, 1].set(cnt).at[:, 2].set(mx)
    smat = smat.at[:, 3].set(mn)

    return _make_expand(n, erows, interpret)(smat, coefs)


def kernel(pcd, W, b, gamma, beta):
    return _edge_conv(pcd, W, b, gamma, beta, _N, _K, _ROWS, _NPAD, 1000)


# speedup vs baseline: 12.7368x; 1.2859x over previous
import functools

import jax
import jax.numpy as jnp
from jax import lax
from jax.experimental import pallas as pl
from jax.experimental.pallas import tpu as pltpu

_N = 10000
_K = 20
_ROWS = 200
_NPAD = 10112


def _knn_stats_body(rows, k, npad, q_ref, post_ref, sq_ref, stats_ref, d_ref):
    i = pl.program_id(0)
    q = q_ref[...]
    post = post_ref[...]
    sq = sq_ref[...]
    inf = jnp.inf

    bmat = jnp.dot(q, post, preferred_element_type=jnp.float32)
    d_ref[...] = (q[:, 3:4] + bmat) + sq
    fq = q[:, 4:5]

    nchunks = npad // 128
    init = tuple(jnp.full((rows, 128), inf, jnp.float32) for _ in range(4))

    def chunk_body(c, carry):
        a1, a2, a3, a4 = carry
        v = d_ref[:, pl.ds(pl.multiple_of(c * 128, 128), 128)]
        a4 = jnp.minimum(a4, v)
        a3, a4 = jnp.minimum(a3, a4), jnp.maximum(a3, a4)
        a2, a3 = jnp.minimum(a2, a3), jnp.maximum(a2, a3)
        a1, a2 = jnp.minimum(a1, a2), jnp.maximum(a1, a2)
        return a1, a2, a3, a4

    a1, a2, a3, a4 = lax.fori_loop(0, nchunks, chunk_body, init)
    cand = jnp.concatenate([a1, a2, a3, a4], axis=1)

    iota_c = lax.broadcasted_iota(jnp.int32, (rows, 512), 1)

    def extract_c(_, carry):
        cc, _ = carry
        m = jnp.min(cc, axis=1, keepdims=True)
        im = jnp.min(jnp.where(cc == m, iota_c, 512), axis=1, keepdims=True)
        return jnp.where(iota_c == im, inf, cc), m

    _, t = lax.fori_loop(0, k, extract_c,
                         (cand, jnp.zeros((rows, 1), jnp.float32)))

    d = d_ref[...]
    le = d <= t
    c_le = jnp.sum(jnp.where(le, 1.0, 0.0), axis=1)
    good = jnp.all(c_le == float(k))

    @pl.when(i == 0)
    def _():
        stats_ref[...] = jnp.concatenate([
            jnp.zeros((2, npad), jnp.float32),
            jnp.full((1, npad), -jnp.inf, jnp.float32),
            jnp.full((1, npad), jnp.inf, jnp.float32),
            jnp.zeros((4, npad), jnp.float32),
        ], axis=0)

    def accum(sel):
        cnt = jnp.sum(jnp.where(sel, 1.0, 0.0), axis=0, keepdims=True)
        sfq = jnp.sum(jnp.where(sel, fq, 0.0), axis=0, keepdims=True)
        mx = jnp.max(jnp.where(sel, fq, -inf), axis=0, keepdims=True)
        mn = jnp.min(jnp.where(sel, fq, inf), axis=0, keepdims=True)
        stats_ref[0:1, :] = stats_ref[0:1, :] + cnt
        stats_ref[1:2, :] = stats_ref[1:2, :] + sfq
        stats_ref[2:3, :] = jnp.maximum(stats_ref[2:3, :], mx)
        stats_ref[3:4, :] = jnp.minimum(stats_ref[3:4, :], mn)

    @pl.when(good)
    def _():
        accum(le)

    @pl.when(jnp.logical_not(good))
    def _():
        iota = lax.broadcasted_iota(jnp.int32, (rows, npad), 1)

        def extract(_, dcur):
            im = jnp.argmin(dcur, axis=1).astype(jnp.int32).reshape(rows, 1)
            return jnp.where(iota == im, inf, dcur)

        dm = lax.fori_loop(0, k, extract, d)
        accum(dm == inf)


def _make_knn_stats(n, k, rows, npad, interpret=False):
    body = functools.partial(_knn_stats_body, rows, k, npad)
    return pl.pallas_call(
        body,
        grid=(n // rows,),
        in_specs=[
            pl.BlockSpec((rows, 8), lambda i: (i, 0)),
            pl.BlockSpec((8, npad), lambda i: (0, 0)),
            pl.BlockSpec((1, npad), lambda i: (0, 0)),
        ],
        out_specs=pl.BlockSpec((8, npad), lambda i: (0, 0)),
        out_shape=jax.ShapeDtypeStruct((8, npad), jnp.float32),
        scratch_shapes=[pltpu.VMEM((rows, npad), jnp.float32)],
        interpret=interpret,
    )


def _expand_body(s_ref, c_ref, o_ref):
    f = s_ref[:, 0:1]
    cnt = s_ref[:, 1:2]
    mx = s_ref[:, 2:3]
    mn = s_ref[:, 3:4]
    pc = c_ref[0:1, :]
    qc = c_ref[1:2, :]
    rc = c_ref[2:3, :]
    t = f * pc + rc + jnp.where(qc >= 0.0, qc * mx, qc * mn)
    y = jnp.where(t >= 0.0, t, 0.2 * t)
    o_ref[...] = jnp.where(cnt > 0.5, y, -jnp.inf)


def _make_expand(n, rows, interpret=False):
    return pl.pallas_call(
        _expand_body,
        grid=(n // rows,),
        in_specs=[
            pl.BlockSpec((rows, 8), lambda i: (i, 0)),
            pl.BlockSpec((8, 64), lambda i: (0, 0)),
        ],
        out_specs=pl.BlockSpec((rows, 64), lambda i: (i, 0)),
        out_shape=jax.ShapeDtypeStruct((n, 64), jnp.float32),
        interpret=interpret,
    )


def _edge_conv(pcd, W, b, gamma, beta, n, k, rows, npad, erows,
               interpret=False):
    f32 = jnp.float32
    pos = pcd[:, :3].astype(f32)
    f = pcd[:, 3].astype(f32)
    qsq = jnp.sum(pos * pos, axis=1)

    qmat = jnp.zeros((n, 8), f32)
    qmat = qmat.at[:, :3].set(pos).at[:, 3].set(qsq).at[:, 4].set(f)
    post = jnp.zeros((8, npad), f32).at[:3, :n].set(-2.0 * pos.T)
    sq = jnp.full((1, npad), 1e30, f32).at[0, :n].set(qsq)

    stats = _make_knn_stats(n, k, rows, npad, interpret)(qmat, post, sq)
    cnt = stats[0, :n]
    sfq = stats[1, :n]
    mx = stats[2, :n]
    mn = stats[3, :n]

    e = float(n * k)
    sum_fp = jnp.dot(cnt, f)
    sum_fp2 = jnp.dot(cnt, f * f)
    sum_fpfq = jnp.dot(sfq, f)
    sum_fq = float(k) * jnp.sum(f)
    sum_fq2 = float(k) * jnp.sum(f * f)
    m_fp = sum_fp / e
    m_fq = sum_fq / e
    v_fp = sum_fp2 / e - m_fp * m_fp
    v_fq = sum_fq2 / e - m_fq * m_fq
    c_pq = sum_fpfq / e - m_fp * m_fq

    w0 = W[0].astype(f32)
    w1 = W[1].astype(f32)
    a0 = w0 - w1
    var = a0 * a0 * v_fp + 2.0 * a0 * w1 * c_pq + w1 * w1 * v_fq
    sigma = jnp.sqrt(var + 1e-5)
    g = gamma.astype(f32) / sigma
    pcoef = a0 * g
    qcoef = w1 * g
    rcoef = beta.astype(f32) - (m_fp * a0 + m_fq * w1) * g

    coefs = jnp.zeros((8, 64), f32)
    coefs = coefs.at[0].set(pcoef).at[1].set(qcoef).at[2].set(rcoef)
    smat = jnp.zeros((n, 8), f32)
    smat = smat.at[:, 0].set(f).at[:, 1].set(cnt).at[:, 2].set(mx)
    smat = smat.at[:, 3].set(mn)

    return _make_expand(n, erows, interpret)(smat, coefs)


def kernel(pcd, W, b, gamma, beta):
    return _edge_conv(pcd, W, b, gamma, beta, _N, _K, _ROWS, _NPAD, 1000)
